# Initial kernel scaffold; baseline (speedup 1.0000x reference)
import jax
import jax.numpy as jnp
from jax.experimental import pallas as pl


def kernel(entity_embedding, relation_embedding, pos_h, pos_r, pos_t, neg_h, neg_t):
    raise NotImplementedError("write your pallas kernel here")


# speedup vs baseline: 1.1196x; 1.1196x over previous
import functools

import jax
import jax.numpy as jnp
from jax import lax
from jax.experimental import pallas as pl
from jax.experimental.pallas import tpu as pltpu
from jax.experimental.pallas import tpu_sc as plsc

D = 64
B = 16384
MARGIN = 1.0

NC = 2
NS = 16
NW = NC * NS
CH = 128

NIDX = 4 * B
E_PER_W = NIDX // NW
R_PER_W = B // NW
E_CHUNKS = E_PER_W // CH
R_CHUNKS = R_PER_W // CH

BLK = 2048
GRID = B // BLK


def _gather_rows(ent, rel, eidx2d, ridx2d):
    mesh = plsc.VectorSubcoreMesh(core_axis_name="core", subcore_axis_name="subcore")

    @functools.partial(
        pl.kernel,
        out_type=(
            jax.ShapeDtypeStruct((NIDX, D), jnp.float32),
            jax.ShapeDtypeStruct((B, D), jnp.float32),
        ),
        mesh=mesh,
        scratch_types=[
            pltpu.VMEM((E_CHUNKS, CH), jnp.int32),
            pltpu.VMEM((R_CHUNKS, CH), jnp.int32),
            pltpu.VMEM((CH, D), jnp.float32),
            pltpu.SemaphoreType.DMA,
        ],
        compiler_params=pltpu.CompilerParams(use_tc_tiling_on_sc=False),
    )
    def gk(ent_hbm, rel_hbm, eidx_hbm, ridx_hbm, ent_out, rel_out,
           eidx_v, ridx_v, rows_v, sem):
        wid = lax.axis_index("subcore") * NC + lax.axis_index("core")
        pltpu.sync_copy(eidx_hbm.at[pl.ds(wid * E_CHUNKS, E_CHUNKS)], eidx_v)
        pltpu.sync_copy(ridx_hbm.at[pl.ds(wid * R_CHUNKS, R_CHUNKS)], ridx_v)
        ebase = wid * E_PER_W
        rbase = wid * R_PER_W

        @pl.loop(0, E_CHUNKS)
        def _(c):
            pltpu.async_copy(ent_hbm.at[eidx_v.at[c]], rows_v, sem).wait()
            pltpu.sync_copy(rows_v, ent_out.at[pl.ds(ebase + c * CH, CH)])

        @pl.loop(0, R_CHUNKS)
        def _(c):
            pltpu.async_copy(rel_hbm.at[ridx_v.at[c]], rows_v, sem).wait()
            pltpu.sync_copy(rows_v, rel_out.at[pl.ds(rbase + c * CH, CH)])

    return gk(ent, rel, eidx2d, ridx2d)


def _loss_body(h_ref, t_ref, hn_ref, tn_ref, r_ref, out_ref):
    i = pl.program_id(0)

    def nrm(x):
        n = jnp.sqrt(jnp.sum(x * x, axis=1, keepdims=True))
        return x / (n + 1e-12)

    h = nrm(h_ref[...])
    t = nrm(t_ref[...])
    hn = nrm(hn_ref[...])
    tn = nrm(tn_ref[...])
    r = r_ref[...]
    pos = jnp.sum(jnp.abs(h + r - t), axis=1)
    neg = jnp.sum(jnp.abs(hn + r - tn), axis=1)
    part = jnp.sum(jnp.maximum(MARGIN + pos - neg, 0.0)) * (1.0 / B)

    @pl.when(i == 0)
    def _():
        out_ref[...] = jnp.zeros_like(out_ref)

    out_ref[...] += jnp.reshape(part, (1, 1))


def kernel(entity_embedding, relation_embedding, pos_h, pos_r, pos_t, neg_h, neg_t):
    eidx = jnp.concatenate([pos_h, pos_t, neg_h, neg_t]).reshape(NW * E_CHUNKS, CH)
    ridx = pos_r.reshape(NW * R_CHUNKS, CH)
    g_ent, g_rel = _gather_rows(entity_embedding, relation_embedding, eidx, ridx)

    loss = pl.pallas_call(
        _loss_body,
        grid=(GRID,),
        in_specs=[
            pl.BlockSpec((BLK, D), lambda i: (i, 0)),
            pl.BlockSpec((BLK, D), lambda i: (i + GRID, 0)),
            pl.BlockSpec((BLK, D), lambda i: (i + 2 * GRID, 0)),
            pl.BlockSpec((BLK, D), lambda i: (i + 3 * GRID, 0)),
            pl.BlockSpec((BLK, D), lambda i: (i, 0)),
        ],
        out_specs=pl.BlockSpec((1, 1), lambda i: (0, 0)),
        out_shape=jax.ShapeDtypeStruct((1, 1), jnp.float32),
    )(g_ent, g_ent, g_ent, g_ent, g_rel)
    return loss[0, 0]
